# Initial kernel scaffold; baseline (speedup 1.0000x reference)
import jax
import jax.numpy as jnp
from jax.experimental import pallas as pl


def kernel(x, emb_weight):
    raise NotImplementedError("write your pallas kernel here")


# speedup vs baseline: 8.5869x; 8.5869x over previous
import jax
import jax.numpy as jnp
from jax.experimental import pallas as pl


def _body(x_ref, emb_ref, o_ref):
    xb = x_ref[...]
    emb = emb_ref[...]
    t = jnp.concatenate([emb] * 8, axis=1)
    x3 = xb[:, :, None]
    r0 = t[0][None, None, :]
    r1 = t[1][None, None, :]
    r2 = t[2][None, None, :]
    r3 = t[3][None, None, :]
    lo = jnp.where(x3 == 0, r0, r1)
    hi = jnp.where(x3 == 2, r2, r3)
    o_ref[...] = jnp.where(x3 < 2, lo, hi)


def kernel(x, emb_weight):
    B, L = x.shape
    C = 4 * 8
    R = 64
    grid = (B // R,)
    return pl.pallas_call(
        _body,
        grid=grid,
        in_specs=[
            pl.BlockSpec((R, L), lambda i: (i, 0)),
            pl.BlockSpec((4, 4), lambda i: (0, 0)),
        ],
        out_specs=pl.BlockSpec((R, L, C), lambda i: (i, 0, 0)),
        out_shape=jax.ShapeDtypeStruct((B, L, C), jnp.float32),
    )(x, emb_weight)
